# baseline (speedup 1.0000x reference)
import jax
import jax.numpy as jnp
from jax.experimental import pallas as pl

B, N, D = 64, 8192, 64


def _z_kernel(x_ref, p_ref, z_ref):
    z_ref[0, 0, :] = jnp.dot(x_ref[0], p_ref[...])[:, 0]


def _compute_z(x, projection):
    z3 = pl.pallas_call(
        _z_kernel,
        grid=(B,),
        in_specs=[
            pl.BlockSpec((1, N, D), lambda b: (b, 0, 0)),
            pl.BlockSpec((D, 1), lambda b: (0, 0)),
        ],
        out_specs=pl.BlockSpec((1, 1, N), lambda b: (b, 0, 0)),
        out_shape=jax.ShapeDtypeStruct((B, 1, N), jnp.float32),
    )(x, projection)
    return z3[:, 0, :]


def kernel(x, projection):
    z = _compute_z(x, projection)
    idx = jnp.argsort(z, axis=1)
    return jnp.take_along_axis(x, idx[:, :, None], axis=1)
